# baseline (speedup 1.0000x reference)
import functools

import jax
import jax.numpy as jnp
from jax import lax
from jax.experimental import pallas as pl
from jax.experimental.pallas import tpu as pltpu
from jax.experimental.pallas import tpu_sc as plsc

NC = 2
NS = 16
NW = NC * NS
K = 80


def _count_body(ns, n_nodes, r_rel, epw, nrpad,
                src_h, dst_h, typ_h, zer_h,
                cnt_h, gidx_h, seg_h,
                src_c, dst_c, typ_c, gi_c, sg_c, idxb, ones_b, cnt_sp, sem):
    del sem
    cid = lax.axis_index("c")
    sid = lax.axis_index("s")
    wid = cid * ns + sid
    base = wid * epw
    pltpu.sync_copy(src_h.at[pl.ds(base, epw)], src_c)
    pltpu.sync_copy(dst_h.at[pl.ds(base, epw)], dst_c)
    pltpu.sync_copy(typ_h.at[pl.ds(base, epw)], typ_c)
    sl = nrpad // ns
    pltpu.sync_copy(zer_h.at[pl.ds(sid * sl, sl)], cnt_sp.at[pl.ds(sid * sl, sl)])
    for g in range(K // 16):
        ones_b[pl.ds(16 * g, 16)] = jnp.ones((16,), jnp.float32)

    def gs_body(g, carry):
        st = pl.multiple_of(g * 16, 16)
        s = src_c[pl.ds(st, 16)]
        t = typ_c[pl.ds(st, 16)]
        d = dst_c[pl.ds(st, 16)]
        gi_c[pl.ds(st, 16)] = t * n_nodes + s
        sg_c[pl.ds(st, 16)] = d * r_rel + t
        return carry

    lax.fori_loop(0, epw // 16, gs_body, 0)
    pltpu.sync_copy(gi_c, gidx_h.at[pl.ds(base, epw)])
    pltpu.sync_copy(sg_c, seg_h.at[pl.ds(base, epw)])
    plsc.subcore_barrier()

    def blk(b, carry):
        for g in range(K // 16):
            st = pl.multiple_of(b * K + 16 * g, 16)
            idxb[pl.ds(16 * g, 16)] = sg_c[pl.ds(st, 16)]
        pltpu.sync_copy(ones_b, cnt_sp.at[idxb], add=True)
        return carry

    lax.fori_loop(0, epw // K, blk, 0)
    plsc.subcore_barrier()
    pltpu.sync_copy(cnt_sp.at[pl.ds(sid * sl, sl)],
                    cnt_h.at[cid, pl.ds(sid * sl, sl)])


def _agg_body(ns, epw, nrpad, n_nodes, hw, zrows,
              tab_h, gidx_h, seg_h, inv_h, zer_h,
              acc_h,
              gi_c, sg_c,
              idxg0, idxg1, idxs0, idxs1, idxd0, idxd1, wb0, wb1,
              rows0, rows1,
              acc_sp, inv_sp,
              sem_w0, sem_w1, sem_r0, sem_r1, sem_c0, sem_c1):
    cid = lax.axis_index("c")
    sid = lax.axis_index("s")
    wid = cid * ns + sid
    base = wid * epw
    pltpu.sync_copy(gidx_h.at[pl.ds(base, epw)], gi_c)
    pltpu.sync_copy(seg_h.at[pl.ds(base, epw)], sg_c)
    sl = nrpad // ns
    pltpu.sync_copy(inv_h.at[pl.ds(sid * sl, sl)], inv_sp.at[pl.ds(sid * sl, sl)])
    nz = n_nodes // zrows

    @pl.when(sid < nz)
    def _zero():
        pltpu.sync_copy(zer_h.at[pl.ds(sid * zrows, zrows), :],
                        acc_sp.at[pl.ds(sid * zrows, zrows), :])

    plsc.subcore_barrier()

    idxg = (idxg0, idxg1)
    idxs = (idxs0, idxs1)
    idxd = (idxd0, idxd1)
    wb = (wb0, wb1)
    rows = (rows0, rows1)
    sem_w = (sem_w0, sem_w1)
    sem_r = (sem_r0, sem_r1)
    sem_c = (sem_c0, sem_c1)
    nb = epw // K

    def build_and_fire(blk, s):
        for g in range(K // 16):
            st = pl.multiple_of(blk * K + 16 * g, 16)
            idxg[s][pl.ds(16 * g, 16)] = gi_c[pl.ds(st, 16)]
            sg = sg_c[pl.ds(st, 16)]
            idxs[s][pl.ds(16 * g, 16)] = sg
            idxd[s][pl.ds(16 * g, 16)] = lax.shift_right_logical(sg, 3)
        pltpu.make_async_copy(inv_sp.at[idxs[s]], wb[s], sem_w[s]).start()
        pltpu.make_async_copy(tab_h.at[idxg[s]], rows[s], sem_r[s]).start()

    def scale_and_scatter(s):
        pltpu.make_async_copy(inv_sp.at[idxs[s]], wb[s], sem_w[s]).wait()
        pltpu.make_async_copy(tab_h.at[idxg[s]], rows[s], sem_r[s]).wait()
        for g in range(K // 16):
            wv = wb[s][pl.ds(16 * g, 16)]
            for l in range(16):
                k = 16 * g + l
                w = jnp.full((16,), wv[l], jnp.float32)
                for j in range(hw // 16):
                    rows[s][k, pl.ds(16 * j, 16)] = (
                        rows[s][k, pl.ds(16 * j, 16)] * w)
        pltpu.make_async_copy(rows[s], acc_sp.at[idxd[s]],
                              sem_c[s]).start(add=True)

    def wait_scatter(s):
        pltpu.make_async_copy(rows[s], acc_sp.at[idxd[s]], sem_c[s]).wait()

    build_and_fire(0, 0)
    build_and_fire(1, 1)

    def body(i, carry):
        scale_and_scatter(0)
        scale_and_scatter(1)
        wait_scatter(0)
        build_and_fire(2 * i + 2, 0)
        wait_scatter(1)
        build_and_fire(2 * i + 3, 1)
        return carry

    lax.fori_loop(0, (nb - 3) // 2, body, 0)
    scale_and_scatter(0)
    scale_and_scatter(1)
    wait_scatter(0)
    build_and_fire(nb - 1, 0)
    wait_scatter(1)
    scale_and_scatter(0)
    wait_scatter(0)

    plsc.subcore_barrier()

    @pl.when(sid < nz)
    def _writeout():
        pltpu.sync_copy(acc_sp.at[pl.ds(sid * zrows, zrows), :],
                        acc_h.at[cid, pl.ds(sid * zrows, zrows), :])


def _tca_body(nblk, cnt_ref, x_ref, w1_ref, r1_ref, b1_ref,
              xw_ref, hb_ref, inv_ref):
    i = pl.program_id(0)
    r = pl.program_id(1)
    xw_ref[...] = jnp.dot(x_ref[...], w1_ref[0],
                          preferred_element_type=jnp.float32)

    @pl.when(r == 0)
    def _():
        hb_ref[...] = (jnp.dot(x_ref[...], r1_ref[...],
                               preferred_element_type=jnp.float32)
                       + b1_ref[...])

    @pl.when((r == 0) & (i == 0))
    def _():
        c = cnt_ref[0] + cnt_ref[1]
        inv_ref[...] = 1.0 / jnp.maximum(c, 1.0)


def _tcb_body(acc_ref, hb1_ref, w2_ref, r2_ref, b2_ref, hw2_ref, hb2_ref):
    r = pl.program_id(1)
    h = jnp.maximum(acc_ref[0] + acc_ref[1] + hb1_ref[...], 0.0)
    hw2_ref[...] = jnp.dot(h, w2_ref[0], preferred_element_type=jnp.float32)

    @pl.when(r == 0)
    def _():
        hb2_ref[...] = (jnp.dot(h, r2_ref[...],
                                preferred_element_type=jnp.float32)
                        + b2_ref[...])


def _tcc_body(acc_ref, hb2_ref, out_ref):
    out_ref[...] = acc_ref[0] + acc_ref[1] + hb2_ref[...]


def kernel(x, edge_index, edge_type, W1, root1, b1, W2, root2, b2):
    n_nodes, d_in = x.shape
    r_rel, _, h_dim = W1.shape
    c_out = W2.shape[2]
    e_edges = edge_type.shape[0]
    assert r_rel == 8 and e_edges % (NW * K) == 0 and n_nodes % 1000 == 0

    epw = e_edges // NW
    nr = n_nodes * r_rel
    nrpad = ((nr + 2047) // 2048) * 2048
    c_pad = 128
    zrows = 1000

    src = edge_index[0]
    dst = edge_index[1]
    mesh = plsc.VectorSubcoreMesh(core_axis_name="c", subcore_axis_name="s",
                                  num_cores=NC, num_subcores=NS)

    count_k = pl.kernel(
        functools.partial(_count_body, NS, n_nodes, r_rel, epw, nrpad),
        out_type=[
            jax.ShapeDtypeStruct((NC, nrpad), jnp.float32),
            jax.ShapeDtypeStruct((e_edges,), jnp.int32),
            jax.ShapeDtypeStruct((e_edges,), jnp.int32),
        ],
        mesh=mesh,
        scratch_types=[
            pltpu.VMEM((epw,), jnp.int32),
            pltpu.VMEM((epw,), jnp.int32),
            pltpu.VMEM((epw,), jnp.int32),
            pltpu.VMEM((epw,), jnp.int32),
            pltpu.VMEM((epw,), jnp.int32),
            pltpu.VMEM((K,), jnp.int32),
            pltpu.VMEM((K,), jnp.float32),
            pltpu.VMEM_SHARED((nrpad,), jnp.float32),
            pltpu.SemaphoreType.DMA,
        ],
    )
    zeros_nr = jnp.zeros((nrpad,), jnp.float32)
    cnts, gidx, seg = count_k(src, dst, edge_type, zeros_nr)

    nblk = 25
    brows = n_nodes // nblk
    tca = pl.pallas_call(
        functools.partial(_tca_body, nblk),
        grid=(nblk, r_rel),
        in_specs=[
            pl.BlockSpec((NC, nrpad // 128, 128), lambda i, r: (0, 0, 0)),
            pl.BlockSpec((brows, d_in), lambda i, r: (i, 0)),
            pl.BlockSpec((1, d_in, h_dim), lambda i, r: (r, 0, 0)),
            pl.BlockSpec((d_in, h_dim), lambda i, r: (0, 0)),
            pl.BlockSpec((1, h_dim), lambda i, r: (0, 0)),
        ],
        out_specs=[
            pl.BlockSpec((brows, h_dim), lambda i, r: (r * nblk + i, 0)),
            pl.BlockSpec((brows, h_dim), lambda i, r: (i, 0)),
            pl.BlockSpec((nrpad // 128, 128), lambda i, r: (0, 0)),
        ],
        out_shape=[
            jax.ShapeDtypeStruct((r_rel * n_nodes, h_dim), jnp.float32),
            jax.ShapeDtypeStruct((n_nodes, h_dim), jnp.float32),
            jax.ShapeDtypeStruct((nrpad // 128, 128), jnp.float32),
        ],
    )
    xw1, hb1, inv = tca(cnts.reshape(NC, nrpad // 128, 128), x, W1,
                        root1, b1.reshape(1, h_dim))
    inv = inv.reshape(nrpad)

    def make_agg(width):
        return pl.kernel(
            functools.partial(_agg_body, NS, epw, nrpad, n_nodes, width,
                              zrows),
            out_type=[
                jax.ShapeDtypeStruct((NC, n_nodes, width), jnp.float32),
            ],
            mesh=mesh,
            scratch_types=(
                [pltpu.VMEM((epw,), jnp.int32) for _ in range(2)]
                + [pltpu.VMEM((K,), jnp.int32) for _ in range(6)]
                + [pltpu.VMEM((K,), jnp.float32) for _ in range(2)]
                + [pltpu.VMEM((K, width), jnp.float32) for _ in range(2)]
                + [pltpu.VMEM_SHARED((n_nodes, width), jnp.float32),
                   pltpu.VMEM_SHARED((nrpad,), jnp.float32)]
                + [pltpu.SemaphoreType.DMA for _ in range(6)]
            ),
        )

    zeros_h = jnp.zeros((n_nodes, h_dim), jnp.float32)
    [acc1] = make_agg(h_dim)(xw1, gidx, seg, inv, zeros_h)

    w2p = jnp.pad(W2, ((0, 0), (0, 0), (0, c_pad - c_out)))
    r2p = jnp.pad(root2, ((0, 0), (0, c_pad - c_out)))
    b2p = jnp.pad(b2, (0, c_pad - c_out)).reshape(1, c_pad)
    tcb = pl.pallas_call(
        _tcb_body,
        grid=(nblk, r_rel),
        in_specs=[
            pl.BlockSpec((NC, brows, h_dim), lambda i, r: (0, i, 0)),
            pl.BlockSpec((brows, h_dim), lambda i, r: (i, 0)),
            pl.BlockSpec((1, h_dim, c_pad), lambda i, r: (r, 0, 0)),
            pl.BlockSpec((h_dim, c_pad), lambda i, r: (0, 0)),
            pl.BlockSpec((1, c_pad), lambda i, r: (0, 0)),
        ],
        out_specs=[
            pl.BlockSpec((brows, c_pad), lambda i, r: (r * nblk + i, 0)),
            pl.BlockSpec((brows, c_pad), lambda i, r: (i, 0)),
        ],
        out_shape=[
            jax.ShapeDtypeStruct((r_rel * n_nodes, c_pad), jnp.float32),
            jax.ShapeDtypeStruct((n_nodes, c_pad), jnp.float32),
        ],
    )
    hw2, hb2 = tcb(acc1, hb1, w2p, r2p, b2p)

    zeros_c = jnp.zeros((n_nodes, c_pad), jnp.float32)
    [acc2] = make_agg(c_pad)(hw2, gidx, seg, inv, zeros_c)

    tcc = pl.pallas_call(
        _tcc_body,
        grid=(nblk,),
        in_specs=[
            pl.BlockSpec((NC, brows, c_pad), lambda i: (0, i, 0)),
            pl.BlockSpec((brows, c_pad), lambda i: (i, 0)),
        ],
        out_specs=pl.BlockSpec((brows, c_pad), lambda i: (i, 0)),
        out_shape=jax.ShapeDtypeStruct((n_nodes, c_pad), jnp.float32),
    )
    res = tcc(acc2, hb2)
    return res[:, :c_out]


# speedup vs baseline: 1.0580x; 1.0580x over previous
import functools

import jax
import jax.numpy as jnp
from jax import lax
from jax.experimental import pallas as pl
from jax.experimental.pallas import tpu as pltpu
from jax.experimental.pallas import tpu_sc as plsc

NC = 2
NS = 16
NW = NC * NS
K = 80


def _count_body(ns, n_nodes, r_rel, epw, nrpad,
                src_h, dst_h, typ_h, zer_h,
                cnt_h, gidx_h, seg_h,
                src_c, dst_c, typ_c, gi_c, sg_c, idxb, ones_b, cnt_sp, sem):
    del sem
    cid = lax.axis_index("c")
    sid = lax.axis_index("s")
    wid = cid * ns + sid
    base = wid * epw
    pltpu.sync_copy(src_h.at[pl.ds(base, epw)], src_c)
    pltpu.sync_copy(dst_h.at[pl.ds(base, epw)], dst_c)
    pltpu.sync_copy(typ_h.at[pl.ds(base, epw)], typ_c)
    sl = nrpad // ns
    pltpu.sync_copy(zer_h.at[pl.ds(sid * sl, sl)], cnt_sp.at[pl.ds(sid * sl, sl)])
    for g in range(K // 16):
        ones_b[pl.ds(16 * g, 16)] = jnp.ones((16,), jnp.float32)

    def gs_body(g, carry):
        st = pl.multiple_of(g * 16, 16)
        s = src_c[pl.ds(st, 16)]
        t = typ_c[pl.ds(st, 16)]
        d = dst_c[pl.ds(st, 16)]
        gi_c[pl.ds(st, 16)] = t * n_nodes + s
        sg_c[pl.ds(st, 16)] = d * r_rel + t
        return carry

    lax.fori_loop(0, epw // 16, gs_body, 0)
    pltpu.sync_copy(gi_c, gidx_h.at[pl.ds(base, epw)])
    pltpu.sync_copy(sg_c, seg_h.at[pl.ds(base, epw)])
    plsc.subcore_barrier()

    def blk(b, carry):
        for g in range(K // 16):
            st = pl.multiple_of(b * K + 16 * g, 16)
            idxb[pl.ds(16 * g, 16)] = sg_c[pl.ds(st, 16)]
        pltpu.sync_copy(ones_b, cnt_sp.at[idxb], add=True)
        return carry

    lax.fori_loop(0, epw // K, blk, 0)
    plsc.subcore_barrier()
    pltpu.sync_copy(cnt_sp.at[pl.ds(sid * sl, sl)],
                    cnt_h.at[cid, pl.ds(sid * sl, sl)])


def _agg_body(ns, epw, nrpad, n_nodes, gw, sw, zrows,
              tab_h, gidx_h, seg_h, inv_h, zer_h,
              acc_h,
              gi_c, sg_c,
              idxg0, idxg1, idxs0, idxs1, idxd0, idxd1, wb0, wb1,
              rows0, rows1, comp0, comp1,
              acc_sp, inv_sp,
              sem_w0, sem_w1, sem_r0, sem_r1, sem_c0, sem_c1):
    cid = lax.axis_index("c")
    sid = lax.axis_index("s")
    wid = cid * ns + sid
    base = wid * epw
    pltpu.sync_copy(gidx_h.at[pl.ds(base, epw)], gi_c)
    pltpu.sync_copy(seg_h.at[pl.ds(base, epw)], sg_c)
    sl = nrpad // ns
    pltpu.sync_copy(inv_h.at[pl.ds(sid * sl, sl)], inv_sp.at[pl.ds(sid * sl, sl)])
    nz = n_nodes // zrows

    @pl.when(sid < nz)
    def _zero():
        pltpu.sync_copy(zer_h.at[pl.ds(sid * zrows, zrows), :],
                        acc_sp.at[pl.ds(sid * zrows, zrows), :])

    plsc.subcore_barrier()

    idxg = (idxg0, idxg1)
    idxs = (idxs0, idxs1)
    idxd = (idxd0, idxd1)
    wb = (wb0, wb1)
    rows = (rows0, rows1)
    comp = (comp0, comp1)
    sem_w = (sem_w0, sem_w1)
    sem_r = (sem_r0, sem_r1)
    sem_c = (sem_c0, sem_c1)
    del sem_c
    nb = epw // K

    def build_and_fire(blk, s):
        for g in range(K // 16):
            st = pl.multiple_of(blk * K + 16 * g, 16)
            idxg[s][pl.ds(16 * g, 16)] = gi_c[pl.ds(st, 16)]
            sg = sg_c[pl.ds(st, 16)]
            idxs[s][pl.ds(16 * g, 16)] = sg
            idxd[s][pl.ds(16 * g, 16)] = lax.shift_right_logical(sg, 3)
        pltpu.make_async_copy(inv_sp.at[idxs[s]], wb[s], sem_w[s]).start()
        pltpu.make_async_copy(tab_h.at[idxg[s]], rows[s], sem_r[s]).start()

    def consume(s):
        pltpu.make_async_copy(inv_sp.at[idxs[s]], wb[s], sem_w[s]).wait()
        pltpu.make_async_copy(tab_h.at[idxg[s]], rows[s], sem_r[s]).wait()
        out_buf = rows[s] if sw == gw else comp[s]
        for g in range(K // 16):
            wv = wb[s][pl.ds(16 * g, 16)]
            for l in range(16):
                k = 16 * g + l
                w = jnp.full((16,), wv[l], jnp.float32)
                for j in range(sw // 16):
                    out_buf[k, pl.ds(16 * j, 16)] = (
                        rows[s][k, pl.ds(16 * j, 16)] * w)
        pltpu.sync_copy(out_buf, acc_sp.at[idxd[s]], add=True)

    build_and_fire(0, 0)
    build_and_fire(1, 1)

    def body(i, carry):
        consume(0)
        build_and_fire(2 * i + 2, 0)
        consume(1)
        build_and_fire(2 * i + 3, 1)
        return carry

    lax.fori_loop(0, (nb - 3) // 2, body, 0)
    consume(0)
    build_and_fire(nb - 1, 0)
    consume(1)
    consume(0)

    plsc.subcore_barrier()

    @pl.when(sid < nz)
    def _writeout():
        pltpu.sync_copy(acc_sp.at[pl.ds(sid * zrows, zrows), :],
                        acc_h.at[cid, pl.ds(sid * zrows, zrows), :])


def _tca_body(nblk, cnt_ref, x_ref, w1_ref, r1_ref, b1_ref,
              xw_ref, hb_ref, inv_ref):
    i = pl.program_id(0)
    r = pl.program_id(1)
    xw_ref[...] = jnp.dot(x_ref[...], w1_ref[0],
                          preferred_element_type=jnp.float32)

    @pl.when(r == 0)
    def _():
        hb_ref[...] = (jnp.dot(x_ref[...], r1_ref[...],
                               preferred_element_type=jnp.float32)
                       + b1_ref[...])

    @pl.when((r == 0) & (i == 0))
    def _():
        c = cnt_ref[0] + cnt_ref[1]
        inv_ref[...] = 1.0 / jnp.maximum(c, 1.0)


def _tcb_body(acc_ref, hb1_ref, w2_ref, r2_ref, b2_ref, hw2_ref, hb2_ref):
    r = pl.program_id(1)
    h = jnp.maximum(acc_ref[0] + acc_ref[1] + hb1_ref[...], 0.0)
    hw2_ref[...] = jnp.dot(h, w2_ref[0], preferred_element_type=jnp.float32)

    @pl.when(r == 0)
    def _():
        hb2_ref[...] = (jnp.dot(h, r2_ref[...],
                                preferred_element_type=jnp.float32)
                        + b2_ref[...])


def _tcc_body(acc_ref, hb2_ref, out_ref):
    out_ref[...] = acc_ref[0] + acc_ref[1] + hb2_ref[...]


def kernel(x, edge_index, edge_type, W1, root1, b1, W2, root2, b2):
    n_nodes, d_in = x.shape
    r_rel, _, h_dim = W1.shape
    c_out = W2.shape[2]
    e_edges = edge_type.shape[0]
    assert r_rel == 8 and e_edges % (NW * K) == 0 and n_nodes % 1000 == 0

    epw = e_edges // NW
    nr = n_nodes * r_rel
    nrpad = ((nr + 2047) // 2048) * 2048
    c_pad = 128
    c_sc = 48
    zrows = 1000

    src = edge_index[0]
    dst = edge_index[1]
    mesh = plsc.VectorSubcoreMesh(core_axis_name="c", subcore_axis_name="s",
                                  num_cores=NC, num_subcores=NS)

    count_k = pl.kernel(
        functools.partial(_count_body, NS, n_nodes, r_rel, epw, nrpad),
        out_type=[
            jax.ShapeDtypeStruct((NC, nrpad), jnp.float32),
            jax.ShapeDtypeStruct((e_edges,), jnp.int32),
            jax.ShapeDtypeStruct((e_edges,), jnp.int32),
        ],
        mesh=mesh,
        scratch_types=[
            pltpu.VMEM((epw,), jnp.int32),
            pltpu.VMEM((epw,), jnp.int32),
            pltpu.VMEM((epw,), jnp.int32),
            pltpu.VMEM((epw,), jnp.int32),
            pltpu.VMEM((epw,), jnp.int32),
            pltpu.VMEM((K,), jnp.int32),
            pltpu.VMEM((K,), jnp.float32),
            pltpu.VMEM_SHARED((nrpad,), jnp.float32),
            pltpu.SemaphoreType.DMA,
        ],
    )
    zeros_nr = jnp.zeros((nrpad,), jnp.float32)
    cnts, gidx, seg = count_k(src, dst, edge_type, zeros_nr)

    nblk = 25
    brows = n_nodes // nblk
    tca = pl.pallas_call(
        functools.partial(_tca_body, nblk),
        grid=(nblk, r_rel),
        in_specs=[
            pl.BlockSpec((NC, nrpad // 128, 128), lambda i, r: (0, 0, 0)),
            pl.BlockSpec((brows, d_in), lambda i, r: (i, 0)),
            pl.BlockSpec((1, d_in, h_dim), lambda i, r: (r, 0, 0)),
            pl.BlockSpec((d_in, h_dim), lambda i, r: (0, 0)),
            pl.BlockSpec((1, h_dim), lambda i, r: (0, 0)),
        ],
        out_specs=[
            pl.BlockSpec((brows, h_dim), lambda i, r: (r * nblk + i, 0)),
            pl.BlockSpec((brows, h_dim), lambda i, r: (i, 0)),
            pl.BlockSpec((nrpad // 128, 128), lambda i, r: (0, 0)),
        ],
        out_shape=[
            jax.ShapeDtypeStruct((r_rel * n_nodes, h_dim), jnp.float32),
            jax.ShapeDtypeStruct((n_nodes, h_dim), jnp.float32),
            jax.ShapeDtypeStruct((nrpad // 128, 128), jnp.float32),
        ],
    )
    xw1, hb1, inv = tca(cnts.reshape(NC, nrpad // 128, 128), x, W1,
                        root1, b1.reshape(1, h_dim))
    inv = inv.reshape(nrpad)

    def make_agg(gw, sw):
        comp_shape = (K, sw) if sw != gw else (16,)
        return pl.kernel(
            functools.partial(_agg_body, NS, epw, nrpad, n_nodes, gw, sw,
                              zrows),
            out_type=[
                jax.ShapeDtypeStruct((NC, n_nodes, sw), jnp.float32),
            ],
            mesh=mesh,
            scratch_types=(
                [pltpu.VMEM((epw,), jnp.int32) for _ in range(2)]
                + [pltpu.VMEM((K,), jnp.int32) for _ in range(6)]
                + [pltpu.VMEM((K,), jnp.float32) for _ in range(2)]
                + [pltpu.VMEM((K, gw), jnp.float32) for _ in range(2)]
                + [pltpu.VMEM(comp_shape, jnp.float32) for _ in range(2)]
                + [pltpu.VMEM_SHARED((n_nodes, sw), jnp.float32),
                   pltpu.VMEM_SHARED((nrpad,), jnp.float32)]
                + [pltpu.SemaphoreType.DMA for _ in range(6)]
            ),
        )

    zeros_h = jnp.zeros((n_nodes, h_dim), jnp.float32)
    [acc1] = make_agg(h_dim, h_dim)(xw1, gidx, seg, inv, zeros_h)

    w2p = jnp.pad(W2, ((0, 0), (0, 0), (0, c_pad - c_out)))
    r2p = jnp.pad(root2, ((0, 0), (0, c_pad - c_out)))
    b2p = jnp.pad(b2, (0, c_pad - c_out)).reshape(1, c_pad)
    tcb = pl.pallas_call(
        _tcb_body,
        grid=(nblk, r_rel),
        in_specs=[
            pl.BlockSpec((NC, brows, h_dim), lambda i, r: (0, i, 0)),
            pl.BlockSpec((brows, h_dim), lambda i, r: (i, 0)),
            pl.BlockSpec((1, h_dim, c_pad), lambda i, r: (r, 0, 0)),
            pl.BlockSpec((h_dim, c_pad), lambda i, r: (0, 0)),
            pl.BlockSpec((1, c_pad), lambda i, r: (0, 0)),
        ],
        out_specs=[
            pl.BlockSpec((brows, c_pad), lambda i, r: (r * nblk + i, 0)),
            pl.BlockSpec((brows, c_pad), lambda i, r: (i, 0)),
        ],
        out_shape=[
            jax.ShapeDtypeStruct((r_rel * n_nodes, c_pad), jnp.float32),
            jax.ShapeDtypeStruct((n_nodes, c_pad), jnp.float32),
        ],
    )
    hw2, hb2 = tcb(acc1, hb1, w2p, r2p, b2p)

    zeros_c = jnp.zeros((n_nodes, c_pad), jnp.float32)
    [acc2] = make_agg(c_pad, c_pad)(hw2, gidx, seg, inv, zeros_c)

    tcc = pl.pallas_call(
        _tcc_body,
        grid=(nblk,),
        in_specs=[
            pl.BlockSpec((NC, brows, c_pad), lambda i: (0, i, 0)),
            pl.BlockSpec((brows, c_pad), lambda i: (i, 0)),
        ],
        out_specs=pl.BlockSpec((brows, c_pad), lambda i: (i, 0)),
        out_shape=jax.ShapeDtypeStruct((n_nodes, c_pad), jnp.float32),
    )
    res = tcc(acc2, hb2)
    return res[:, :c_out]


# speedup vs baseline: 1.0608x; 1.0026x over previous
import functools

import jax
import jax.numpy as jnp
from jax import lax
from jax.experimental import pallas as pl
from jax.experimental.pallas import tpu as pltpu
from jax.experimental.pallas import tpu_sc as plsc

NC = 2
NS = 16
NW = NC * NS
K = 80


def _count_body(ns, n_nodes, r_rel, epw, nrpad,
                src_h, dst_h, typ_h, zer_h,
                cnt_h, gidx_h, seg_h,
                src_c, dst_c, typ_c, gi_c, sg_c, idxb, ones_b, cnt_sp, sem):
    del sem
    cid = lax.axis_index("c")
    sid = lax.axis_index("s")
    wid = cid * ns + sid
    base = wid * epw
    pltpu.sync_copy(src_h.at[pl.ds(base, epw)], src_c)
    pltpu.sync_copy(dst_h.at[pl.ds(base, epw)], dst_c)
    pltpu.sync_copy(typ_h.at[pl.ds(base, epw)], typ_c)
    sl = nrpad // ns
    pltpu.sync_copy(zer_h.at[pl.ds(sid * sl, sl)], cnt_sp.at[pl.ds(sid * sl, sl)])
    for g in range(K // 16):
        ones_b[pl.ds(16 * g, 16)] = jnp.ones((16,), jnp.float32)

    def gs_body(g, carry):
        st = pl.multiple_of(g * 16, 16)
        s = src_c[pl.ds(st, 16)]
        t = typ_c[pl.ds(st, 16)]
        d = dst_c[pl.ds(st, 16)]
        gi_c[pl.ds(st, 16)] = t * n_nodes + s
        sg_c[pl.ds(st, 16)] = d * r_rel + t
        return carry

    lax.fori_loop(0, epw // 16, gs_body, 0)
    pltpu.sync_copy(gi_c, gidx_h.at[pl.ds(base, epw)])
    pltpu.sync_copy(sg_c, seg_h.at[pl.ds(base, epw)])
    plsc.subcore_barrier()

    def blk(b, carry):
        for g in range(K // 16):
            st = pl.multiple_of(b * K + 16 * g, 16)
            idxb[pl.ds(16 * g, 16)] = sg_c[pl.ds(st, 16)]
        pltpu.sync_copy(ones_b, cnt_sp.at[idxb], add=True)
        return carry

    lax.fori_loop(0, epw // K, blk, 0)
    plsc.subcore_barrier()
    pltpu.sync_copy(cnt_sp.at[pl.ds(sid * sl, sl)],
                    cnt_h.at[cid, pl.ds(sid * sl, sl)])


def _agg_body(ns, epw, nrpad, n_nodes, gw, sw, zrows,
              tab_h, gidx_h, seg_h, inv_h, zer_h,
              acc_h,
              gi_c, sg_c,
              idxg0, idxg1, idxs0, idxs1, idxd0, idxd1, wb0, wb1,
              rows0, rows1, comp0, comp1,
              acc_sp, inv_sp,
              sem_w0, sem_w1, sem_r0, sem_r1, sem_c0, sem_c1):
    cid = lax.axis_index("c")
    sid = lax.axis_index("s")
    wid = cid * ns + sid
    base = wid * epw
    pltpu.sync_copy(gidx_h.at[pl.ds(base, epw)], gi_c)
    pltpu.sync_copy(seg_h.at[pl.ds(base, epw)], sg_c)
    sl = nrpad // ns
    pltpu.sync_copy(inv_h.at[pl.ds(sid * sl, sl)], inv_sp.at[pl.ds(sid * sl, sl)])
    nz = n_nodes // zrows

    @pl.when(sid < nz)
    def _zero():
        pltpu.sync_copy(zer_h.at[pl.ds(sid * zrows, zrows), :],
                        acc_sp.at[pl.ds(sid * zrows, zrows), :])

    plsc.subcore_barrier()

    idxg = (idxg0, idxg1)
    idxs = (idxs0, idxs1)
    idxd = (idxd0, idxd1)
    wb = (wb0, wb1)
    rows = (rows0, rows1)
    comp = (comp0, comp1)
    sem_w = (sem_w0, sem_w1)
    sem_r = (sem_r0, sem_r1)
    sem_c = (sem_c0, sem_c1)
    del sem_c
    nb = epw // K

    def build_and_fire(blk, s):
        for g in range(K // 16):
            st = pl.multiple_of(blk * K + 16 * g, 16)
            idxg[s][pl.ds(16 * g, 16)] = gi_c[pl.ds(st, 16)]
            sg = sg_c[pl.ds(st, 16)]
            idxs[s][pl.ds(16 * g, 16)] = sg
            idxd[s][pl.ds(16 * g, 16)] = lax.shift_right_logical(sg, 3)
        pltpu.make_async_copy(inv_sp.at[idxs[s]], wb[s], sem_w[s]).start()
        pltpu.make_async_copy(tab_h.at[idxg[s]], rows[s], sem_r[s]).start()

    def consume(s):
        pltpu.make_async_copy(inv_sp.at[idxs[s]], wb[s], sem_w[s]).wait()
        pltpu.make_async_copy(tab_h.at[idxg[s]], rows[s], sem_r[s]).wait()
        out_buf = rows[s] if sw == gw else comp[s]
        for g in range(K // 16):
            wv = wb[s][pl.ds(16 * g, 16)]
            for l in range(16):
                k = 16 * g + l
                w = jnp.full((16,), wv[l], jnp.float32)
                for j in range(sw // 16):
                    out_buf[k, pl.ds(16 * j, 16)] = (
                        rows[s][k, pl.ds(16 * j, 16)] * w)
        pltpu.sync_copy(out_buf, acc_sp.at[idxd[s]], add=True)

    build_and_fire(0, 0)
    build_and_fire(1, 1)

    def body(i, carry):
        consume(0)
        build_and_fire(2 * i + 2, 0)
        consume(1)
        build_and_fire(2 * i + 3, 1)
        return carry

    lax.fori_loop(0, (nb - 3) // 2, body, 0)
    consume(0)
    build_and_fire(nb - 1, 0)
    consume(1)
    consume(0)

    plsc.subcore_barrier()

    @pl.when(sid < nz)
    def _writeout():
        pltpu.sync_copy(acc_sp.at[pl.ds(sid * zrows, zrows), :],
                        acc_h.at[cid, pl.ds(sid * zrows, zrows), :])


def _tca_body(nblk, cnt_ref, x_ref, w1_ref, r1_ref, b1_ref,
              xw_ref, hb_ref, inv_ref):
    i = pl.program_id(0)
    r = pl.program_id(1)
    xw_ref[...] = jnp.dot(x_ref[...], w1_ref[0],
                          preferred_element_type=jnp.float32)

    @pl.when(r == 0)
    def _():
        hb_ref[...] = (jnp.dot(x_ref[...], r1_ref[...],
                               preferred_element_type=jnp.float32)
                       + b1_ref[...][None, :])

    @pl.when((r == 0) & (i == 0))
    def _():
        c = cnt_ref[0] + cnt_ref[1]
        inv_ref[...] = 1.0 / jnp.maximum(c, 1.0)


def _tcb_body(c_pad, acc_ref, hb1_ref, w2_ref, r2_ref, b2_ref,
              hw2_ref, hb2_ref):
    r = pl.program_id(1)
    c_out = w2_ref.shape[2]
    pad = ((0, 0), (0, c_pad - c_out))
    h = jnp.maximum(acc_ref[0] + acc_ref[1] + hb1_ref[...], 0.0)
    hw2_ref[...] = jnp.pad(
        jnp.dot(h, w2_ref[0], preferred_element_type=jnp.float32), pad)

    @pl.when(r == 0)
    def _():
        hb2_ref[...] = jnp.pad(
            jnp.dot(h, r2_ref[...], preferred_element_type=jnp.float32)
            + b2_ref[...][None, :], pad)


def _tcc_body(c_out, acc_ref, hb2_ref, out_ref):
    s = acc_ref[0] + acc_ref[1] + hb2_ref[...]
    out_ref[...] = s[:, :c_out]


def kernel(x, edge_index, edge_type, W1, root1, b1, W2, root2, b2):
    n_nodes, d_in = x.shape
    r_rel, _, h_dim = W1.shape
    c_out = W2.shape[2]
    e_edges = edge_type.shape[0]
    assert r_rel == 8 and e_edges % (NW * K) == 0 and n_nodes % 1000 == 0

    epw = e_edges // NW
    nr = n_nodes * r_rel
    nrpad = ((nr + 2047) // 2048) * 2048
    c_pad = 128
    c_sc = 48
    zrows = 1000

    mesh = plsc.VectorSubcoreMesh(core_axis_name="c", subcore_axis_name="s",
                                  num_cores=NC, num_subcores=NS)

    count_k = pl.kernel(
        functools.partial(_count_body, NS, n_nodes, r_rel, epw, nrpad),
        out_type=[
            jax.ShapeDtypeStruct((NC, nrpad), jnp.float32),
            jax.ShapeDtypeStruct((e_edges,), jnp.int32),
            jax.ShapeDtypeStruct((e_edges,), jnp.int32),
        ],
        mesh=mesh,
        scratch_types=[
            pltpu.VMEM((epw,), jnp.int32),
            pltpu.VMEM((epw,), jnp.int32),
            pltpu.VMEM((epw,), jnp.int32),
            pltpu.VMEM((epw,), jnp.int32),
            pltpu.VMEM((epw,), jnp.int32),
            pltpu.VMEM((K,), jnp.int32),
            pltpu.VMEM((K,), jnp.float32),
            pltpu.VMEM_SHARED((nrpad,), jnp.float32),
            pltpu.SemaphoreType.DMA,
        ],
    )
    zeros_nr = jnp.zeros((nrpad,), jnp.float32)
    cnts, gidx, seg = count_k(edge_index[0], edge_index[1], edge_type,
                              zeros_nr)

    nblk = 25
    brows = n_nodes // nblk
    tca = pl.pallas_call(
        functools.partial(_tca_body, nblk),
        grid=(nblk, r_rel),
        in_specs=[
            pl.BlockSpec((NC, nrpad), lambda i, r: (0, 0)),
            pl.BlockSpec((brows, d_in), lambda i, r: (i, 0)),
            pl.BlockSpec((1, d_in, h_dim), lambda i, r: (r, 0, 0)),
            pl.BlockSpec((d_in, h_dim), lambda i, r: (0, 0)),
            pl.BlockSpec((h_dim,), lambda i, r: (0,)),
        ],
        out_specs=[
            pl.BlockSpec((brows, h_dim), lambda i, r: (r * nblk + i, 0)),
            pl.BlockSpec((brows, h_dim), lambda i, r: (i, 0)),
            pl.BlockSpec((nrpad,), lambda i, r: (0,)),
        ],
        out_shape=[
            jax.ShapeDtypeStruct((r_rel * n_nodes, h_dim), jnp.float32),
            jax.ShapeDtypeStruct((n_nodes, h_dim), jnp.float32),
            jax.ShapeDtypeStruct((nrpad,), jnp.float32),
        ],
    )
    xw1, hb1, inv = tca(cnts, x, W1, root1, b1)

    def make_agg(gw, sw):
        comp_shape = (K, sw) if sw != gw else (16,)
        return pl.kernel(
            functools.partial(_agg_body, NS, epw, nrpad, n_nodes, gw, sw,
                              zrows),
            out_type=[
                jax.ShapeDtypeStruct((NC, n_nodes, sw), jnp.float32),
            ],
            mesh=mesh,
            scratch_types=(
                [pltpu.VMEM((epw,), jnp.int32) for _ in range(2)]
                + [pltpu.VMEM((K,), jnp.int32) for _ in range(6)]
                + [pltpu.VMEM((K,), jnp.float32) for _ in range(2)]
                + [pltpu.VMEM((K, gw), jnp.float32) for _ in range(2)]
                + [pltpu.VMEM(comp_shape, jnp.float32) for _ in range(2)]
                + [pltpu.VMEM_SHARED((n_nodes, sw), jnp.float32),
                   pltpu.VMEM_SHARED((nrpad,), jnp.float32)]
                + [pltpu.SemaphoreType.DMA for _ in range(6)]
            ),
        )

    zeros_h = jnp.zeros((n_nodes, h_dim), jnp.float32)
    [acc1] = make_agg(h_dim, h_dim)(xw1, gidx, seg, inv, zeros_h)

    tcb = pl.pallas_call(
        functools.partial(_tcb_body, c_pad),
        grid=(nblk, r_rel),
        in_specs=[
            pl.BlockSpec((NC, brows, h_dim), lambda i, r: (0, i, 0)),
            pl.BlockSpec((brows, h_dim), lambda i, r: (i, 0)),
            pl.BlockSpec((1, h_dim, c_out), lambda i, r: (r, 0, 0)),
            pl.BlockSpec((h_dim, c_out), lambda i, r: (0, 0)),
            pl.BlockSpec((c_out,), lambda i, r: (0,)),
        ],
        out_specs=[
            pl.BlockSpec((brows, c_pad), lambda i, r: (r * nblk + i, 0)),
            pl.BlockSpec((brows, c_pad), lambda i, r: (i, 0)),
        ],
        out_shape=[
            jax.ShapeDtypeStruct((r_rel * n_nodes, c_pad), jnp.float32),
            jax.ShapeDtypeStruct((n_nodes, c_pad), jnp.float32),
        ],
    )
    hw2, hb2 = tcb(acc1, hb1, W2, root2, b2)

    zeros_c = jnp.zeros((n_nodes, c_pad), jnp.float32)
    [acc2] = make_agg(c_pad, c_pad)(hw2, gidx, seg, inv, zeros_c)

    tcc = pl.pallas_call(
        functools.partial(_tcc_body, c_out),
        grid=(nblk,),
        in_specs=[
            pl.BlockSpec((NC, brows, c_pad), lambda i: (0, i, 0)),
            pl.BlockSpec((brows, c_pad), lambda i: (i, 0)),
        ],
        out_specs=pl.BlockSpec((brows, c_out), lambda i: (i, 0)),
        out_shape=jax.ShapeDtypeStruct((n_nodes, c_out), jnp.float32),
    )
    return tcc(acc2, hb2)


# speedup vs baseline: 1.6347x; 1.5411x over previous
import functools

import jax
import jax.numpy as jnp
from jax import lax
from jax.experimental import pallas as pl
from jax.experimental.pallas import tpu as pltpu
from jax.experimental.pallas import tpu_sc as plsc

NC = 2
NS = 16
NW = NC * NS
K = 80


def _count_body(ns, n_nodes, r_rel, epw, nrpad,
                src_h, dst_h, typ_h, zer_h,
                cnt_h, gidx_h, seg_h,
                src_c, dst_c, typ_c, gi_c, sg_c, idxb, ones_b, cnt_sp, sem):
    del sem
    cid = lax.axis_index("c")
    sid = lax.axis_index("s")
    wid = cid * ns + sid
    base = wid * epw
    pltpu.sync_copy(src_h.at[pl.ds(base, epw)], src_c)
    pltpu.sync_copy(dst_h.at[pl.ds(base, epw)], dst_c)
    pltpu.sync_copy(typ_h.at[pl.ds(base, epw)], typ_c)
    sl = nrpad // ns
    pltpu.sync_copy(zer_h.at[pl.ds(sid * sl, sl)], cnt_sp.at[pl.ds(sid * sl, sl)])
    for g in range(K // 16):
        ones_b[pl.ds(16 * g, 16)] = jnp.ones((16,), jnp.float32)

    def gs_body(g, carry):
        st = pl.multiple_of(g * 16, 16)
        s = src_c[pl.ds(st, 16)]
        t = typ_c[pl.ds(st, 16)]
        d = dst_c[pl.ds(st, 16)]
        gi_c[pl.ds(st, 16)] = t * n_nodes + s
        sg_c[pl.ds(st, 16)] = d * r_rel + t
        return carry

    lax.fori_loop(0, epw // 16, gs_body, 0)
    pltpu.sync_copy(gi_c, gidx_h.at[pl.ds(base, epw)])
    pltpu.sync_copy(sg_c, seg_h.at[pl.ds(base, epw)])
    plsc.subcore_barrier()

    def blk(b, carry):
        for g in range(K // 16):
            st = pl.multiple_of(b * K + 16 * g, 16)
            idxb[pl.ds(16 * g, 16)] = sg_c[pl.ds(st, 16)]
        pltpu.sync_copy(ones_b, cnt_sp.at[idxb], add=True)
        return carry

    lax.fori_loop(0, epw // K, blk, 0)
    plsc.subcore_barrier()
    pltpu.sync_copy(cnt_sp.at[pl.ds(sid * sl, sl)],
                    cnt_h.at[cid, pl.ds(sid * sl, sl)])


def _agg_body(ns, epw, nrpad, n_nodes, gw, sw, zrows,
              tab_h, gidx_h, seg_h, inv_h, zer_h,
              acc_h,
              gi_c, sg_c,
              idxg0, idxg1, idxs0, idxs1, idxd0, idxd1, wb0, wb1,
              rows0, rows1, comp0, comp1,
              acc_sp, inv_sp,
              sem_w0, sem_w1, sem_r0, sem_r1, sem_c0, sem_c1):
    cid = lax.axis_index("c")
    sid = lax.axis_index("s")
    wid = cid * ns + sid
    base = wid * epw
    pltpu.sync_copy(gidx_h.at[pl.ds(base, epw)], gi_c)
    pltpu.sync_copy(seg_h.at[pl.ds(base, epw)], sg_c)
    sl = nrpad // ns
    pltpu.sync_copy(inv_h.at[pl.ds(sid * sl, sl)], inv_sp.at[pl.ds(sid * sl, sl)])
    nz = n_nodes // zrows

    @pl.when(sid < nz)
    def _zero():
        pltpu.sync_copy(zer_h.at[pl.ds(sid * zrows, zrows), :],
                        acc_sp.at[pl.ds(sid * zrows, zrows), :])

    plsc.subcore_barrier()

    idxg = (idxg0, idxg1)
    idxs = (idxs0, idxs1)
    idxd = (idxd0, idxd1)
    wb = (wb0, wb1)
    rows = (rows0, rows1)
    comp = (comp0, comp1)
    sem_w = (sem_w0, sem_w1)
    sem_r = (sem_r0, sem_r1)
    sem_c = (sem_c0, sem_c1)
    del sem_c
    nb = epw // K

    def build_and_fire(blk, s):
        for g in range(K // 16):
            st = pl.multiple_of(blk * K + 16 * g, 16)
            idxg[s][pl.ds(16 * g, 16)] = gi_c[pl.ds(st, 16)]
            sg = sg_c[pl.ds(st, 16)]
            idxs[s][pl.ds(16 * g, 16)] = sg
            idxd[s][pl.ds(16 * g, 16)] = lax.shift_right_logical(sg, 3)
        pltpu.make_async_copy(inv_sp.at[idxs[s]], wb[s], sem_w[s]).start()
        pltpu.make_async_copy(tab_h.at[idxg[s]], rows[s], sem_r[s]).start()

    def consume(s):
        pltpu.make_async_copy(inv_sp.at[idxs[s]], wb[s], sem_w[s]).wait()
        pltpu.make_async_copy(tab_h.at[idxg[s]], rows[s], sem_r[s]).wait()
        out_buf = rows[s] if sw == gw else comp[s]
        for g in range(K // 16):
            wv = wb[s][pl.ds(16 * g, 16)]
            for l in range(16):
                k = 16 * g + l
                w = jnp.full((16,), wv[l], jnp.float32)
                for j in range(sw // 16):
                    out_buf[k, pl.ds(16 * j, 16)] = (
                        rows[s][k, pl.ds(16 * j, 16)] * w)
        pltpu.sync_copy(out_buf, acc_sp.at[idxd[s]], add=True)

    build_and_fire(0, 0)
    build_and_fire(1, 1)

    def body(i, carry):
        consume(0)
        build_and_fire(2 * i + 2, 0)
        consume(1)
        build_and_fire(2 * i + 3, 1)
        return carry

    lax.fori_loop(0, (nb - 3) // 2, body, 0)
    consume(0)
    build_and_fire(nb - 1, 0)
    consume(1)
    consume(0)

    plsc.subcore_barrier()

    @pl.when(sid < nz)
    def _writeout():
        pltpu.sync_copy(acc_sp.at[pl.ds(sid * zrows, zrows), :],
                        acc_h.at[cid, pl.ds(sid * zrows, zrows), :])


def _tca_body(cnt_ref, x_ref, w1_ref, r1_ref, b1_ref,
              xw_ref, hb_ref, inv_ref):
    r = pl.program_id(0)
    xw_ref[...] = jnp.dot(x_ref[...], w1_ref[0],
                          preferred_element_type=jnp.float32)

    @pl.when(r == 0)
    def _():
        hb_ref[...] = (jnp.dot(x_ref[...], r1_ref[...],
                               preferred_element_type=jnp.float32)
                       + b1_ref[...][None, :])
        c = cnt_ref[0] + cnt_ref[1]
        inv_ref[...] = 1.0 / jnp.maximum(c, 1.0)


def _tcb_body(c_pad, acc_ref, hb1_ref, w2_ref, r2_ref, b2_ref,
              hw2_ref, hb2_ref):
    r = pl.program_id(0)
    c_out = w2_ref.shape[2]
    pad = ((0, 0), (0, c_pad - c_out))
    h = jnp.maximum(acc_ref[0] + acc_ref[1] + hb1_ref[...], 0.0)
    hw2_ref[...] = jnp.pad(
        jnp.dot(h, w2_ref[0], preferred_element_type=jnp.float32), pad)

    @pl.when(r == 0)
    def _():
        hb2_ref[...] = jnp.pad(
            jnp.dot(h, r2_ref[...], preferred_element_type=jnp.float32)
            + b2_ref[...][None, :], pad)


def _tcc_body(c_out, acc_ref, hb2_ref, out_ref):
    s = acc_ref[0] + acc_ref[1] + hb2_ref[...]
    out_ref[...] = s[:, :c_out]


def kernel(x, edge_index, edge_type, W1, root1, b1, W2, root2, b2):
    n_nodes, d_in = x.shape
    r_rel, _, h_dim = W1.shape
    c_out = W2.shape[2]
    e_edges = edge_type.shape[0]
    assert r_rel == 8 and e_edges % (NW * K) == 0 and n_nodes % 1000 == 0

    epw = e_edges // NW
    nr = n_nodes * r_rel
    nrpad = ((nr + 2047) // 2048) * 2048
    c_pad = 128
    c_sc = 48
    zrows = 1000

    mesh = plsc.VectorSubcoreMesh(core_axis_name="c", subcore_axis_name="s",
                                  num_cores=NC, num_subcores=NS)

    count_k = pl.kernel(
        functools.partial(_count_body, NS, n_nodes, r_rel, epw, nrpad),
        out_type=[
            jax.ShapeDtypeStruct((NC, nrpad), jnp.float32),
            jax.ShapeDtypeStruct((e_edges,), jnp.int32),
            jax.ShapeDtypeStruct((e_edges,), jnp.int32),
        ],
        mesh=mesh,
        scratch_types=[
            pltpu.VMEM((epw,), jnp.int32),
            pltpu.VMEM((epw,), jnp.int32),
            pltpu.VMEM((epw,), jnp.int32),
            pltpu.VMEM((epw,), jnp.int32),
            pltpu.VMEM((epw,), jnp.int32),
            pltpu.VMEM((K,), jnp.int32),
            pltpu.VMEM((K,), jnp.float32),
            pltpu.VMEM_SHARED((nrpad,), jnp.float32),
            pltpu.SemaphoreType.DMA,
        ],
    )
    zeros_nr = jnp.zeros((nrpad,), jnp.float32)
    cnts, gidx, seg = count_k(edge_index[0], edge_index[1], edge_type,
                              zeros_nr)

    tca = pl.pallas_call(
        _tca_body,
        grid=(r_rel,),
        in_specs=[
            pl.BlockSpec((NC, nrpad), lambda r: (0, 0)),
            pl.BlockSpec((n_nodes, d_in), lambda r: (0, 0)),
            pl.BlockSpec((1, d_in, h_dim), lambda r: (r, 0, 0)),
            pl.BlockSpec((d_in, h_dim), lambda r: (0, 0)),
            pl.BlockSpec((h_dim,), lambda r: (0,)),
        ],
        out_specs=[
            pl.BlockSpec((n_nodes, h_dim), lambda r: (r, 0)),
            pl.BlockSpec((n_nodes, h_dim), lambda r: (0, 0)),
            pl.BlockSpec((nrpad,), lambda r: (0,)),
        ],
        out_shape=[
            jax.ShapeDtypeStruct((r_rel * n_nodes, h_dim), jnp.float32),
            jax.ShapeDtypeStruct((n_nodes, h_dim), jnp.float32),
            jax.ShapeDtypeStruct((nrpad,), jnp.float32),
        ],
    )
    xw1, hb1, inv = tca(cnts, x, W1, root1, b1)

    def make_agg(gw, sw):
        comp_shape = (K, sw) if sw != gw else (16,)
        return pl.kernel(
            functools.partial(_agg_body, NS, epw, nrpad, n_nodes, gw, sw,
                              zrows),
            out_type=[
                jax.ShapeDtypeStruct((NC, n_nodes, sw), jnp.float32),
            ],
            mesh=mesh,
            scratch_types=(
                [pltpu.VMEM((epw,), jnp.int32) for _ in range(2)]
                + [pltpu.VMEM((K,), jnp.int32) for _ in range(6)]
                + [pltpu.VMEM((K,), jnp.float32) for _ in range(2)]
                + [pltpu.VMEM((K, gw), jnp.float32) for _ in range(2)]
                + [pltpu.VMEM(comp_shape, jnp.float32) for _ in range(2)]
                + [pltpu.VMEM_SHARED((n_nodes, sw), jnp.float32),
                   pltpu.VMEM_SHARED((nrpad,), jnp.float32)]
                + [pltpu.SemaphoreType.DMA for _ in range(6)]
            ),
        )

    zeros_h = jnp.zeros((n_nodes, h_dim), jnp.float32)
    [acc1] = make_agg(h_dim, h_dim)(xw1, gidx, seg, inv, zeros_h)

    tcb = pl.pallas_call(
        functools.partial(_tcb_body, c_pad),
        grid=(r_rel,),
        in_specs=[
            pl.BlockSpec((NC, n_nodes, h_dim), lambda r: (0, 0, 0)),
            pl.BlockSpec((n_nodes, h_dim), lambda r: (0, 0)),
            pl.BlockSpec((1, h_dim, c_out), lambda r: (r, 0, 0)),
            pl.BlockSpec((h_dim, c_out), lambda r: (0, 0)),
            pl.BlockSpec((c_out,), lambda r: (0,)),
        ],
        out_specs=[
            pl.BlockSpec((n_nodes, c_pad), lambda r: (r, 0)),
            pl.BlockSpec((n_nodes, c_pad), lambda r: (0, 0)),
        ],
        out_shape=[
            jax.ShapeDtypeStruct((r_rel * n_nodes, c_pad), jnp.float32),
            jax.ShapeDtypeStruct((n_nodes, c_pad), jnp.float32),
        ],
    )
    hw2, hb2 = tcb(acc1, hb1, W2, root2, b2)

    zeros_c = jnp.zeros((n_nodes, c_pad), jnp.float32)
    [acc2] = make_agg(c_pad, c_pad)(hw2, gidx, seg, inv, zeros_c)

    tcc = pl.pallas_call(
        functools.partial(_tcc_body, c_out),
        grid=(1,),
        in_specs=[
            pl.BlockSpec((NC, n_nodes, c_pad), lambda i: (0, 0, 0)),
            pl.BlockSpec((n_nodes, c_pad), lambda i: (0, 0)),
        ],
        out_specs=pl.BlockSpec((n_nodes, c_out), lambda i: (0, 0)),
        out_shape=jax.ShapeDtypeStruct((n_nodes, c_out), jnp.float32),
    )
    return tcc(acc2, hb2)
